# Initial kernel scaffold; baseline (speedup 1.0000x reference)
import jax
import jax.numpy as jnp
from jax.experimental import pallas as pl


def kernel(coords, feats, neighbor_idx, pool_ids1, neighbor_idx1, pool_ids2, neighbor_idx2, W_in, W_f, g_f, b_f, W_d1, W_c1, g_c1, b_c1, W_d2, W_c2, g_c2, b_c2, W_u2, W_dc1, g_dc1, b_dc1, W_u1, W_df, g_df, b_df, g_out, b_out, W_lin, b_lin):
    raise NotImplementedError("write your pallas kernel here")


# speedup vs baseline: 7.2119x; 7.2119x over previous
import functools

import jax
import jax.numpy as jnp
from jax import lax
from jax.experimental import pallas as pl
from jax.experimental.pallas import tpu as pltpu
from jax.experimental.pallas import tpu_sc as plsc

N = 32768
N1 = 4096
N2 = 512
K = 27
M = 32

NC = 2
NS = 16
NW = NC * NS
CH = 128


@functools.cache
def _gather_rows(R, C, Nout, KK):
  rows_w = Nout // NW
  ch = min(CH, rows_w)
  nch = rows_w // ch
  nch_tot = Nout // ch
  mesh = plsc.VectorSubcoreMesh(core_axis_name="c", subcore_axis_name="s")

  @functools.partial(
      pl.kernel,
      out_type=jax.ShapeDtypeStruct((Nout, KK * C), jnp.bfloat16),
      mesh=mesh,
      compiler_params=pltpu.CompilerParams(use_tc_tiling_on_sc=False),
      scratch_types=[
          pltpu.VMEM((KK, ch), jnp.int32),
          pltpu.VMEM((KK, ch, C), jnp.bfloat16),
          pltpu.SemaphoreType.DMA,
          pltpu.SemaphoreType.DMA,
      ],
  )
  def k(h_hbm, j_hbm, out_hbm, idx_v, buf_v, sem, sem2):
    wid = lax.axis_index("s") * NC + lax.axis_index("c")
    base = wid * rows_w
    base_ch = wid * nch

    def chunk_body(i, carry):
      pltpu.sync_copy(j_hbm.at[:, base_ch + i], idx_v)
      cps = [
          pltpu.async_copy(h_hbm.at[idx_v.at[kk]], buf_v.at[kk], sem)
          for kk in range(KK)
      ]
      for cp in cps:
        cp.wait()
      row0 = base + i * ch
      wps = [
          pltpu.async_copy(
              buf_v.at[kk],
              out_hbm.at[pl.ds(row0, ch), pl.ds(kk * C, C)],
              sem2,
          )
          for kk in range(KK)
      ]
      for wp in wps:
        wp.wait()
      return carry

    lax.fori_loop(0, nch, chunk_body, 0)

  def run(h_bf, J):
    return k(h_bf, J.reshape(KK, nch_tot, ch))

  return run


def _sc_gather(h_bf, J):
  KK, Nout = J.shape
  R, C = h_bf.shape
  return _gather_rows(R, C, Nout, KK)(h_bf, J)


@functools.cache
def _bn_relu_bf16(Nl, C):
  def body(x_ref, mu_ref, sc_ref, g_ref, b_ref, o_ref):
    h = ((x_ref[...] - mu_ref[...]) * sc_ref[...]) * g_ref[...] + b_ref[...]
    o_ref[...] = jnp.maximum(h, 0.0).astype(jnp.bfloat16)

  return pl.pallas_call(
      body,
      out_shape=jax.ShapeDtypeStruct((Nl, C), jnp.bfloat16),
  )


def _bn_relu(x, g, b):
  Nl, C = x.shape
  mu = jnp.mean(x, axis=0)
  sc = lax.rsqrt(jnp.var(x, axis=0) + 1e-5)
  return _bn_relu_bf16(Nl, C)(x, mu.reshape(1, C), sc.reshape(1, C),
                              g.reshape(1, C), b.reshape(1, C))


def _sconv(h_bf, J, W):
  Kk, C, D = W.shape
  Nout = J.shape[1]
  gath = _sc_gather(h_bf, J).reshape(Nout, Kk, C)
  return jnp.einsum('nkc,kcd->nd', gath, W.astype(jnp.bfloat16),
                    preferred_element_type=jnp.float32)


def _blocks(x, W, g, b, J):
  for r in range(W.shape[0]):
    h = _bn_relu(x, g[r, 0], b[r, 0])
    h2 = _sconv(h, J, W[r, 0])
    h3 = _bn_relu(h2, g[r, 1], b[r, 1])
    x = x + _sconv(h3, J, W[r, 1])
  return x


def _pool(x, ids, n):
  s = jax.ops.segment_sum(x, ids, num_segments=n)
  c = jax.ops.segment_sum(jnp.ones((x.shape[0], 1), x.dtype), ids,
                          num_segments=n)
  return s / jnp.maximum(c, 1.0)


def kernel(coords, feats, neighbor_idx, pool_ids1, neighbor_idx1, pool_ids2,
           neighbor_idx2, W_in, W_f, g_f, b_f, W_d1, W_c1, g_c1, b_c1, W_d2,
           W_c2, g_c2, b_c2, W_u2, W_dc1, g_dc1, b_dc1, W_u1, W_df, g_df,
           b_df, g_out, b_out, W_lin, b_lin):
  J0 = neighbor_idx.T
  J1 = neighbor_idx1.T
  J2 = neighbor_idx2.T

  fbf = jnp.pad(feats.astype(jnp.bfloat16), ((0, 0), (0, M - 3)))
  g0 = _sc_gather(fbf, J0).reshape(N, K, M)[:, :, :3]
  x = jnp.einsum('nkc,kcd->nd', g0, W_in.astype(jnp.bfloat16),
                 preferred_element_type=jnp.float32)

  x = _blocks(x, W_f, g_f, b_f, J0)
  skip_f = x

  x1 = jnp.dot(_pool(x, pool_ids1, N1).astype(jnp.bfloat16),
               W_d1.astype(jnp.bfloat16), preferred_element_type=jnp.float32)
  x1 = _blocks(x1, W_c1, g_c1, b_c1, J1)
  skip1 = x1

  x2 = jnp.dot(_pool(x1, pool_ids2, N2).astype(jnp.bfloat16),
               W_d2.astype(jnp.bfloat16), preferred_element_type=jnp.float32)
  x2 = _blocks(x2, W_c2, g_c2, b_c2, J2)

  gu2 = _sc_gather(x2.astype(jnp.bfloat16), pool_ids2.reshape(1, N1))
  x1 = skip1 + jnp.dot(gu2, W_u2.astype(jnp.bfloat16),
                       preferred_element_type=jnp.float32)
  x1 = _blocks(x1, W_dc1, g_dc1, b_dc1, J1)

  gu1 = _sc_gather(x1.astype(jnp.bfloat16), pool_ids1.reshape(1, N))
  x = skip_f + jnp.dot(gu1, W_u1.astype(jnp.bfloat16),
                       preferred_element_type=jnp.float32)
  x = _blocks(x, W_df, g_df, b_df, J0)

  h = _bn_relu(x, g_out, b_out)
  y = jnp.dot(h, W_lin.astype(jnp.bfloat16),
              preferred_element_type=jnp.float32) + b_lin
  return y
